# baseline (speedup 1.0000x reference)
import jax
import jax.numpy as jnp
from jax import lax
from jax.experimental import pallas as pl
from jax.experimental.pallas import tpu as pltpu
from jax.experimental.pallas import tpu_sc as plsc

NUM_EMBEDDINGS = 1000000
EMBEDDING_DIM = 32
BATCH = 16384
FIELDS = 26

_B = BATCH * FIELDS
_NW = 32
_PER_W = _B // _NW
_NBUF = 4
_NCHUNK = 32
_CHUNK = _PER_W // _NCHUNK
_NGRP = _CHUNK // 16


def _body(table_hbm, rhalf_hbm, chalf_hbm, out_hbm, idx_v, ch_v, rows_v,
          *sems):
    nc = 2
    wid = lax.axis_index("s") * nc + lax.axis_index("c")
    base = wid * _PER_W
    gsem = sems[:_NBUF]
    ssem = sems[_NBUF:]

    def recombine(c):
        for g in range(_NGRP):
            off = c * _CHUNK + g * 16
            r16 = idx_v[pl.ds(off, 16)]
            c16 = ch_v[pl.ds(off, 16)]
            idx_v[pl.ds(off, 16)] = (r16 << 2) | c16

    def gather(c, buf):
        return pltpu.async_copy(
            table_hbm.at[idx_v.at[pl.ds(c * _CHUNK, _CHUNK)]],
            rows_v.at[buf], gsem[buf])

    def store(c, buf):
        return pltpu.async_copy(
            rows_v.at[buf],
            out_hbm.at[pl.ds(base + c * _CHUNK, _CHUNK)], ssem[buf])

    pltpu.sync_copy(rhalf_hbm.at[pl.ds(base, _PER_W)], idx_v)
    pltpu.sync_copy(chalf_hbm.at[pl.ds(base, _PER_W)], ch_v)

    depth = _NBUF - 1
    pending_g = [None] * _NBUF
    pending_s = [None] * _NBUF
    for c in range(depth):
        recombine(c)
        pending_g[c % _NBUF] = gather(c, c % _NBUF)
    for c in range(_NCHUNK):
        buf = c % _NBUF
        n = c + depth
        if n < _NCHUNK:
            recombine(n)
        pending_g[buf].wait()
        pending_g[buf] = None
        pending_s[buf] = store(c, buf)
        if n < _NCHUNK:
            b2 = n % _NBUF
            if pending_s[b2] is not None:
                pending_s[b2].wait()
                pending_s[b2] = None
            pending_g[b2] = gather(n, b2)
    for s in pending_s:
        if s is not None:
            s.wait()


@jax.jit
def _embed(rhalf, chalf, weight):
    mesh = plsc.VectorSubcoreMesh(core_axis_name="c", subcore_axis_name="s")
    fn = pl.kernel(
        _body,
        out_type=jax.ShapeDtypeStruct((_B, EMBEDDING_DIM), jnp.float32),
        mesh=mesh,
        scratch_types=[
            pltpu.VMEM((_PER_W,), jnp.int32),
            pltpu.VMEM((_PER_W,), jnp.int32),
            pltpu.VMEM((_NBUF, _CHUNK, EMBEDDING_DIM), jnp.float32),
        ] + [pltpu.SemaphoreType.DMA] * (2 * _NBUF),
        compiler_params=pltpu.CompilerParams(use_tc_tiling_on_sc=False),
    )
    return fn(weight, rhalf, chalf)


def kernel(input, weight):
    rhalf = (input >> 2).reshape(-1)
    chalf = (input & 3).reshape(-1)
    out = _embed(rhalf, chalf, weight)
    return out.reshape(BATCH, FIELDS, EMBEDDING_DIM)
